# baseline (speedup 1.0000x reference)
import functools

import jax
import jax.numpy as jnp
from jax import lax
from jax.experimental import pallas as pl
from jax.experimental.pallas import tpu as pltpu
from jax.experimental.pallas import tpu_sc as plsc

N_CLASS = 8
EPS = 1e-07

NC = 2
NS = 16
L = 16
NW = NC * NS

TOTAL = 2 * 128 * 128 * 128
CHUNK = TOTAL // NW
BLK = 16384
NBLK = CHUNK // BLK
NKEY = N_CLASS * N_CLASS
HBINS = NKEY * NKEY


U = 8


@functools.partial(
    pl.kernel,
    out_type=jax.ShapeDtypeStruct((NW, HBINS), jnp.float32),
    mesh=plsc.VectorSubcoreMesh(core_axis_name="c", subcore_axis_name="s"),
    compiler_params=pltpu.CompilerParams(needs_layout_passes=False),
    scratch_types=[
        pltpu.VMEM((BLK,), jnp.float32),
        pltpu.VMEM((BLK,), jnp.float32),
        pltpu.VMEM((BLK,), jnp.int32),
        pltpu.VMEM((BLK,), jnp.int32),
        pltpu.VMEM((HBINS,), jnp.float32),
        pltpu.SemaphoreType.DMA,
        pltpu.SemaphoreType.DMA,
        pltpu.SemaphoreType.DMA,
        pltpu.SemaphoreType.DMA,
    ],
)
def _sc_joint_hist(src_hbm, tgt_hbm, out_hbm, src_a, src_b, tgt_a, tgt_b,
                   hist_v, sem_sa, sem_sb, sem_ta, sem_tb):
    wid = lax.axis_index("s") * NC + lax.axis_index("c")
    base = wid * CHUNK

    zeros = jnp.zeros((L,), jnp.float32)
    for k in range(HBINS // L):
        hist_v[pl.ds(k * L, L)] = zeros

    MAGIC = jnp.float32(8388608.0)
    CB = jnp.full((L,), jnp.int32(0x6A000000))
    ones = jnp.ones((L,), jnp.float32)

    src_bufs = (src_a, src_b)
    tgt_bufs = (tgt_a, tgt_b)
    src_sems = (sem_sa, sem_sb)
    tgt_sems = (sem_ta, sem_tb)

    def run_block(src_v, tgt_v):
        @plsc.parallel_loop(0, BLK // (2 * L), unroll=U)
        def vec_body(i):
            o = i * (2 * L)
            p1 = plsc.bitcast(src_v[pl.ds(o, L)] * 512.0 + MAGIC, jnp.int32)
            p2 = plsc.bitcast(src_v[pl.ds(o + L, L)] * 8.0 + MAGIC, jnp.int32)
            t1 = tgt_v[pl.ds(o, L)]
            t2 = tgt_v[pl.ds(o + L, L)]
            idx = (p1 + t1 * NKEY) + (p2 + t2) + CB
            plsc.addupdate_scatter(hist_v, [idx], ones)

    def start(blk):
        buf = blk % 2
        off = base + blk * BLK
        cs = pltpu.async_copy(src_hbm.at[pl.ds(off, BLK)], src_bufs[buf],
                              src_sems[buf])
        ct = pltpu.async_copy(tgt_hbm.at[pl.ds(off, BLK)], tgt_bufs[buf],
                              tgt_sems[buf])
        return cs, ct

    pending = start(0)
    for blk in range(NBLK):
        buf = blk % 2
        cs, ct = pending
        if blk + 1 < NBLK:
            nxt = start(blk + 1)
        cs.wait()
        ct.wait()
        run_block(src_bufs[buf], tgt_bufs[buf])
        if blk + 1 < NBLK:
            pending = nxt

    pltpu.sync_copy(hist_v, out_hbm.at[wid])


def _tc_dice_finish(h_ref, o_ref):
    x = h_ref[...]
    key = lax.broadcasted_iota(jnp.int32, (NS, HBINS), 1)
    s1 = key // (NKEY * N_CLASS)
    t1 = (key // NKEY) % N_CLASS
    s2 = (key // N_CLASS) % N_CLASS
    t2 = key % N_CLASS
    zero = jnp.zeros((NS, HBINS), jnp.float32)
    one = jnp.ones((NS, HBINS), jnp.float32)
    total = jnp.float32(0.0)
    for b in range(2):
        xb = x[b * NS:(b + 1) * NS, :]
        for c in range(N_CLASS):
            m_i = (jnp.where((s1 == c) & (t1 == c), one, zero)
                   + jnp.where((s2 == c) & (t2 == c), one, zero))
            m_s = (jnp.where(s1 == c, one, zero)
                   + jnp.where(s2 == c, one, zero))
            m_t = (jnp.where(t1 == c, one, zero)
                   + jnp.where(t2 == c, one, zero))
            inter = jnp.sum(xb * m_i)
            sv = jnp.sum(xb * m_s)
            tv = jnp.sum(xb * m_t)
            total += (2.0 * inter + EPS) / (sv + tv + 2.0 * EPS)
    o_ref[0, 0] = -total / N_CLASS


def kernel(source, target):
    src_flat = source.reshape(-1)
    tgt_flat = target.reshape(-1)
    h = _sc_joint_hist(src_flat, tgt_flat)
    loss = pl.pallas_call(
        _tc_dice_finish,
        out_shape=jax.ShapeDtypeStruct((1, 1), jnp.float32),
        in_specs=[pl.BlockSpec(memory_space=pltpu.VMEM)],
        out_specs=pl.BlockSpec(memory_space=pltpu.SMEM),
    )(h)
    return loss[0, 0]


# speedup vs baseline: 1.0748x; 1.0748x over previous
import functools

import jax
import jax.numpy as jnp
from jax import lax
from jax.experimental import pallas as pl
from jax.experimental.pallas import tpu as pltpu
from jax.experimental.pallas import tpu_sc as plsc

N_CLASS = 8
EPS = 1e-07

NC = 2
NS = 16
L = 16
NW = NC * NS

TOTAL = 2 * 128 * 128 * 128
CHUNK = TOTAL // NW
BLK = 16384
NBLK = CHUNK // BLK
NKEY = N_CLASS * N_CLASS
HBINS = NKEY * L


U = 4


@functools.partial(
    pl.kernel,
    out_type=jax.ShapeDtypeStruct((NW, HBINS), jnp.float32),
    mesh=plsc.VectorSubcoreMesh(core_axis_name="c", subcore_axis_name="s"),
    compiler_params=pltpu.CompilerParams(needs_layout_passes=False),
    scratch_types=[
        pltpu.VMEM((BLK,), jnp.float32),
        pltpu.VMEM((BLK,), jnp.float32),
        pltpu.VMEM((BLK,), jnp.int32),
        pltpu.VMEM((BLK,), jnp.int32),
        pltpu.VMEM((HBINS,), jnp.float32),
        pltpu.SemaphoreType.DMA,
        pltpu.SemaphoreType.DMA,
        pltpu.SemaphoreType.DMA,
        pltpu.SemaphoreType.DMA,
    ],
)
def _sc_joint_hist(src_hbm, tgt_hbm, out_hbm, src_a, src_b, tgt_a, tgt_b,
                   hist_v, sem_sa, sem_sb, sem_ta, sem_tb):
    wid = lax.axis_index("s") * NC + lax.axis_index("c")
    base = wid * CHUNK

    zeros = jnp.zeros((L,), jnp.float32)
    for k in range(NKEY):
        hist_v[pl.ds(k * L, L)] = zeros

    BIAS = jnp.int32(0x4B000000)
    MAGIC = jnp.float32(8388608.0)
    lane_b = lax.iota(jnp.int32, L) - BIAS
    ones = jnp.ones((L,), jnp.float32)

    src_bufs = (src_a, src_b)
    tgt_bufs = (tgt_a, tgt_b)
    src_sems = (sem_sa, sem_sb)
    tgt_sems = (sem_ta, sem_tb)

    def run_block(src_v, tgt_v):
        @plsc.parallel_loop(0, BLK // L, unroll=U)
        def vec_body(i):
            sp = plsc.bitcast(src_v[pl.ds(i * L, L)] * 128.0 + MAGIC,
                              jnp.int32)
            t = tgt_v[pl.ds(i * L, L)]
            idx = sp + (t * L + lane_b)
            plsc.addupdate_scatter(hist_v, [idx], ones)

    def start(blk):
        buf = blk % 2
        off = base + blk * BLK
        cs = pltpu.async_copy(src_hbm.at[pl.ds(off, BLK)], src_bufs[buf],
                              src_sems[buf])
        ct = pltpu.async_copy(tgt_hbm.at[pl.ds(off, BLK)], tgt_bufs[buf],
                              tgt_sems[buf])
        return cs, ct

    pending = start(0)
    for blk in range(NBLK):
        buf = blk % 2
        cs, ct = pending
        if blk + 1 < NBLK:
            nxt = start(blk + 1)
        cs.wait()
        ct.wait()
        run_block(src_bufs[buf], tgt_bufs[buf])
        if blk + 1 < NBLK:
            pending = nxt

    pltpu.sync_copy(hist_v, out_hbm.at[wid])


def _tc_dice_finish(h_ref, o_ref):
    x = h_ref[...]
    key = lax.broadcasted_iota(jnp.int32, (NS, HBINS), 1) // L
    s_cls = key // N_CLASS
    t_cls = key - s_cls * N_CLASS
    zero = jnp.zeros((NS, HBINS), jnp.float32)
    total = jnp.float32(0.0)
    for b in range(2):
        xb = x[b * NS:(b + 1) * NS, :]
        for c in range(N_CLASS):
            inter = jnp.sum(jnp.where(key == c * (N_CLASS + 1), xb, zero))
            sv = jnp.sum(jnp.where(s_cls == c, xb, zero))
            tv = jnp.sum(jnp.where(t_cls == c, xb, zero))
            total += (2.0 * inter + EPS) / (sv + tv + 2.0 * EPS)
    o_ref[0, 0] = -total / N_CLASS


def kernel(source, target):
    src_flat = source.reshape(-1)
    tgt_flat = target.reshape(-1)
    h = _sc_joint_hist(src_flat, tgt_flat)
    loss = pl.pallas_call(
        _tc_dice_finish,
        out_shape=jax.ShapeDtypeStruct((1, 1), jnp.float32),
        in_specs=[pl.BlockSpec(memory_space=pltpu.VMEM)],
        out_specs=pl.BlockSpec(memory_space=pltpu.SMEM),
    )(h)
    return loss[0, 0]
